# baseline (speedup 1.0000x reference)
import functools

import jax
import jax.numpy as jnp
from jax import lax
from jax.experimental import pallas as pl
from jax.experimental.pallas import tpu as pltpu
from jax.experimental.pallas import tpu_sc as plsc

_MAXD = 2048
_N = 4096
_T = 2 * _MAXD - 1
_EREV = 8192
_NW = 32
_ROWS = _N // _NW
_BLK = 8
_NBLK = _ROWS // _BLK
_NBUF = 3
_GSH = 6144
_NPH = 8
_BIGW = 8336


def _body(g_hbm, idx_hbm, out_hbm, gsh, rowbuf, idxv, ldsem, stsem, mirsem):
    sid = lax.axis_index("s")
    wid = lax.axis_index("c") * 16 + sid
    i0 = wid * _ROWS
    start_min = (_N - _ROWS) - i0

    pltpu.sync_copy(idx_hbm.at[wid], idxv)

    colbase = (1 - lax.axis_index("c")) * (_N // 2)
    per_ph = _NBLK // _NPH

    lds, sts = [], []

    def _scatter(b):
        return pltpu.async_copy(
            rowbuf.at[b % _NBUF], out_hbm.at[idxv.at[b]], stsem
        )

    for phase in range(_NPH):

        @pl.when(sid < per_ph)
        def _load_phase():
            pltpu.async_copy(
                g_hbm.at[phase * per_ph + sid, :, pl.ds(colbase, _GSH)],
                gsh.at[sid],
                mirsem,
            ).wait()

        plsc.subcore_barrier()
        for bb in range(per_ph):
            b = phase * per_ph + bb
            if b >= _NBUF:
                sts[b - _NBUF].wait()
            lds.append(
                pltpu.async_copy(
                    gsh.at[bb, :, pl.ds(start_min - colbase, _N)],
                    rowbuf.at[b % _NBUF],
                    ldsem,
                )
            )
            if b >= 1 and len(sts) == b - 1:
                lds[b - 1].wait()
                sts.append(_scatter(b - 1))
        if phase < _NPH - 1:
            last = (phase + 1) * per_ph - 1
            lds[last].wait()
            sts.append(_scatter(last))
            plsc.subcore_barrier()
    lds[_NBLK - 1].wait()
    sts.append(_scatter(_NBLK - 1))
    for h in sts[_NBLK - _NBUF:]:
        h.wait()


def _toeplitz_rows(g, idx):
    mesh = plsc.VectorSubcoreMesh(core_axis_name="c", subcore_axis_name="s")
    f = functools.partial(
        pl.kernel,
        mesh=mesh,
        out_type=jax.ShapeDtypeStruct((_N, _N), jnp.float32),
        scratch_types=[
            pltpu.VMEM_SHARED((_NBLK // _NPH, _BLK, _GSH), jnp.float32),
            pltpu.VMEM((_NBUF, _BLK, _N), jnp.float32),
            pltpu.VMEM((_NBLK, _BLK), jnp.int32),
            pltpu.SemaphoreType.DMA,
            pltpu.SemaphoreType.DMA,
            pltpu.SemaphoreType.DMA,
        ],
        compiler_params=pltpu.CompilerParams(
            disable_bounds_checks=True,
            disable_semaphore_checks=True,
        ),
    )(_body)
    return f(g, idx)


def kernel(bias_table, query_len, cond_len):
    d = jnp.asarray(query_len, jnp.int32) - jnp.asarray(cond_len, jnp.int32)
    base = jnp.pad(jnp.flip(bias_table), (_N, _N), mode="edge")
    dc = jnp.clip(d, -1900, 1900)
    big = lax.dynamic_slice(base, (_MAXD - dc,), (_BIGW,))
    g8r = jnp.stack([big[7 - gg : 7 - gg + _BIGW - 8] for gg in range(_BLK)])
    gm = jnp.stack(
        [g8r[:, 24 - 8 * b0 : 24 - 8 * b0 + _EREV + 96] for b0 in range(4)]
    )
    g = jnp.stack(
        [gm[:, :, 96 - 32 * b1 : 96 - 32 * b1 + _EREV] for b1 in range(4)]
    ).reshape(_NBLK, _BLK, _EREV)
    idx = jnp.arange(_N, dtype=jnp.int32).reshape(_NW, _NBLK, _BLK)
    return _toeplitz_rows(g, idx)


# speedup vs baseline: 1.1316x; 1.1316x over previous
import functools

import jax
import jax.numpy as jnp
from jax import lax
from jax.experimental import pallas as pl
from jax.experimental.pallas import tpu as pltpu
from jax.experimental.pallas import tpu_sc as plsc

_MAXD = 2048
_N = 4096
_T = 2 * _MAXD - 1
_EREV = 8192
_NW = 32
_ROWS = _N // _NW
_BLK = 8
_NBLK = _ROWS // _BLK
_NBUF = 3
_GSH = 6144
_NPH = 4
_BIGW = 8336


def _body(g_hbm, out_hbm, gsh, rowbuf, ldsem, stsem, mirsem):
    sid = lax.axis_index("s")
    wid = lax.axis_index("c") * 16 + sid
    i0 = wid * _ROWS
    start_min = (_N - _ROWS) - i0

    colbase = (1 - lax.axis_index("c")) * (_N // 2)
    per_ph = _NBLK // _NPH

    lds, sts = [], []

    def _scatter(b):
        return pltpu.async_copy(
            rowbuf.at[b % _NBUF],
            out_hbm.at[pl.ds(i0 + _BLK * b, _BLK), :],
            stsem,
        )

    for phase in range(_NPH):

        @pl.when(sid < per_ph)
        def _load_phase():
            pltpu.async_copy(
                g_hbm.at[phase * per_ph + sid, :, pl.ds(colbase, _GSH)],
                gsh.at[sid],
                mirsem,
            ).wait()

        plsc.subcore_barrier()
        for bb in range(per_ph):
            b = phase * per_ph + bb
            if b >= _NBUF:
                sts[b - _NBUF].wait()
            lds.append(
                pltpu.async_copy(
                    gsh.at[bb, :, pl.ds(start_min - colbase, _N)],
                    rowbuf.at[b % _NBUF],
                    ldsem,
                )
            )
            if b >= 1 and len(sts) == b - 1:
                lds[b - 1].wait()
                sts.append(_scatter(b - 1))
        if phase < _NPH - 1:
            last = (phase + 1) * per_ph - 1
            lds[last].wait()
            sts.append(_scatter(last))
            plsc.subcore_barrier()
    lds[_NBLK - 1].wait()
    sts.append(_scatter(_NBLK - 1))
    for h in sts[_NBLK - _NBUF:]:
        h.wait()


def _toeplitz_rows(g):
    mesh = plsc.VectorSubcoreMesh(core_axis_name="c", subcore_axis_name="s")
    f = functools.partial(
        pl.kernel,
        mesh=mesh,
        out_type=jax.ShapeDtypeStruct((_N, _N), jnp.float32),
        scratch_types=[
            pltpu.VMEM_SHARED((_NBLK // _NPH, _BLK, _GSH), jnp.float32),
            pltpu.VMEM((_NBUF, _BLK, _N), jnp.float32),
            pltpu.SemaphoreType.DMA,
            pltpu.SemaphoreType.DMA,
            pltpu.SemaphoreType.DMA,
        ],
        compiler_params=pltpu.CompilerParams(
            disable_bounds_checks=True,
            disable_semaphore_checks=True,
        ),
    )(_body)
    return f(g)


def kernel(bias_table, query_len, cond_len):
    d = jnp.asarray(query_len, jnp.int32) - jnp.asarray(cond_len, jnp.int32)
    base = jnp.pad(jnp.flip(bias_table), (_N, _N), mode="edge")
    dc = jnp.clip(d, -1900, 1900)
    big = lax.dynamic_slice(base, (_MAXD - dc,), (_BIGW,))
    g8r = jnp.stack([big[7 - gg : 7 - gg + _BIGW - 8] for gg in range(_BLK)])
    gm = jnp.stack(
        [g8r[:, 24 - 8 * b0 : 24 - 8 * b0 + _EREV + 96] for b0 in range(4)]
    )
    g = jnp.stack(
        [gm[:, :, 96 - 32 * b1 : 96 - 32 * b1 + _EREV] for b1 in range(4)]
    ).reshape(_NBLK, _BLK, _EREV)
    return _toeplitz_rows(g)
